# baseline (speedup 1.0000x reference)
import jax
import jax.numpy as jnp
from jax import lax
from jax.experimental import pallas as pl
from jax.experimental.pallas import tpu as pltpu

_EMBED_DIM = 256
_IMG = 1024


def _fused_body(coords_ref, lab_ref, bg_ref, fg_ref, nm_ref,
                sparse_ref, dense_ref):
    b = pl.program_id(0)
    h = pl.program_id(1)

    @pl.when((b == 0) & (h == 1))
    def _sparse():
        B, N, D = sparse_ref.shape
        half = D // 2
        d_idx = lax.broadcasted_iota(jnp.int32, (1, D), 1)
        freq = jnp.exp2((d_idx >> 1).astype(jnp.float32) / half) * jnp.pi
        phase = (d_idx & 1).astype(jnp.float32) * (jnp.pi / 2)
        row = lax.broadcasted_iota(jnp.int32, (2, D), 0)
        w = jnp.where(row == 0, jnp.broadcast_to(freq, (2, D)), 0.0)

        cm = coords_ref[...].reshape(B * N, 2)
        args = jnp.dot(cm, w, preferred_element_type=jnp.float32,
                       precision=lax.Precision.HIGHEST) + phase
        pe = jnp.sin(args).reshape(B, N, D)
        lab = lab_ref[...]
        emb = jnp.where(lab[:, :, None] >= 1,
                        fg_ref[...].reshape(1, 1, D),
                        bg_ref[...].reshape(1, 1, D))
        sparse_ref[...] = pe + emb

    nm_col = nm_ref[...].T.reshape(1, nm_ref.shape[1], 1, 1)
    dense_ref[...] = jnp.broadcast_to(nm_col, dense_ref.shape)


def kernel(coords, labels, point_embed_bg, point_embed_fg, no_mask_embed):
    B, N, _ = coords.shape
    D = _EMBED_DIM
    HW = _IMG // 4

    H_BLK = 64
    sparse, dense = pl.pallas_call(
        _fused_body,
        grid=(B, HW // H_BLK),
        compiler_params=pltpu.CompilerParams(
            dimension_semantics=("parallel", "parallel"),
            allow_input_fusion=[True, False, False, False, False]),
        in_specs=[
            pl.BlockSpec((B, N, 2), lambda b, h: (0, 0, 0)),
            pl.BlockSpec((B, N), lambda b, h: (0, 0)),
            pl.BlockSpec((1, D), lambda b, h: (0, 0)),
            pl.BlockSpec((1, D), lambda b, h: (0, 0)),
            pl.BlockSpec((1, D), lambda b, h: (0, 0)),
        ],
        out_specs=[
            pl.BlockSpec((B, N, D), lambda b, h: (0, 0, 0)),
            pl.BlockSpec((1, D, H_BLK, HW), lambda b, h: (b, 0, h, 0)),
        ],
        out_shape=[
            jax.ShapeDtypeStruct((B, N, D), jnp.float32),
            jax.ShapeDtypeStruct((B, D, HW, HW), jnp.float32),
        ],
    )(coords, labels, point_embed_bg, point_embed_fg, no_mask_embed)
    return (sparse, dense)


# speedup vs baseline: 1.0049x; 1.0049x over previous
import jax
import jax.numpy as jnp
from jax import lax
from jax.experimental import pallas as pl
from jax.experimental.pallas import tpu as pltpu

_EMBED_DIM = 256
_IMG = 1024


def _fused_body(coords_ref, lab_ref, bg_ref, fg_ref, nm_ref,
                sparse_ref, dense_ref, nm_col_ref):
    b = pl.program_id(0)
    h = pl.program_id(1)

    @pl.when((b == 0) & (h == 1))
    def _sparse():
        B, N, D = sparse_ref.shape
        half = D // 2
        d_idx = lax.broadcasted_iota(jnp.int32, (1, D), 1)
        freq = jnp.exp2((d_idx >> 1).astype(jnp.float32) / half) * jnp.pi
        phase = (d_idx & 1).astype(jnp.float32) * (jnp.pi / 2)
        row = lax.broadcasted_iota(jnp.int32, (2, D), 0)
        w = jnp.where(row == 0, jnp.broadcast_to(freq, (2, D)), 0.0)

        cm = coords_ref[...].reshape(B * N, 2)
        args = jnp.dot(cm, w, preferred_element_type=jnp.float32,
                       precision=lax.Precision.HIGHEST) + phase
        pe = jnp.sin(args).reshape(B, N, D)
        lab = lab_ref[...]
        emb = jnp.where(lab[:, :, None] >= 1,
                        fg_ref[...].reshape(1, 1, D),
                        bg_ref[...].reshape(1, 1, D))
        sparse_ref[...] = pe + emb

    @pl.when((b == 0) & (h == 0))
    def _init():
        nm_col_ref[...] = nm_ref[...].T

    d_blk = dense_ref.shape[1]
    nm_blk = nm_col_ref[pl.ds(h * d_blk, d_blk), :]
    nm_col = nm_blk.reshape(1, d_blk, 1, 1)
    dense_ref[...] = jnp.broadcast_to(nm_col, dense_ref.shape)


def kernel(coords, labels, point_embed_bg, point_embed_fg, no_mask_embed):
    B, N, _ = coords.shape
    D = _EMBED_DIM
    HW = _IMG // 4

    D_BLK = 32
    sparse, dense = pl.pallas_call(
        _fused_body,
        grid=(B, D // D_BLK),
        compiler_params=pltpu.CompilerParams(
            dimension_semantics=("parallel", "parallel")),
        in_specs=[
            pl.BlockSpec((B, N, 2), lambda b, h: (0, 0, 0)),
            pl.BlockSpec((B, N), lambda b, h: (0, 0)),
            pl.BlockSpec((1, D), lambda b, h: (0, 0)),
            pl.BlockSpec((1, D), lambda b, h: (0, 0)),
            pl.BlockSpec((1, D), lambda b, h: (0, 0)),
        ],
        out_specs=[
            pl.BlockSpec((B, N, D), lambda b, h: (0, 0, 0)),
            pl.BlockSpec((1, D_BLK, HW, HW), lambda b, h: (b, h, 0, 0)),
        ],
        out_shape=[
            jax.ShapeDtypeStruct((B, N, D), jnp.float32),
            jax.ShapeDtypeStruct((B, D, HW, HW), jnp.float32),
        ],
        scratch_shapes=[pltpu.VMEM((D, 1), jnp.float32)],
    )(coords, labels, point_embed_bg, point_embed_fg, no_mask_embed)
    return (sparse, dense)
